# baseline (speedup 1.0000x reference)
import functools

import jax
import jax.numpy as jnp
from jax.experimental import pallas as pl
from jax.experimental.pallas import tpu as pltpu

_NUM_EXPERTS = 64
_EPAD = 128
_HIDDEN = 2048
_TOP_K = 8
_TOKENS = 16384
_BT = 512

_NEG_INF = float("-inf")


def _tree_row_sumsq(x):
    x2 = x * x
    parts = [x2[:, k * 128:(k + 1) * 128] for k in range(x.shape[1] // 128)]
    while len(parts) > 1:
        parts = [parts[2 * j] + parts[2 * j + 1]
                 for j in range(len(parts) // 2)]
    return jnp.sum(parts[0], axis=1, keepdims=True)


def _proto_norm_block(pt_ref, pnt_ref):
    pt = pt_ref[...]
    pnorm = jnp.sqrt(jnp.sum(pt * pt, axis=0, keepdims=True))
    pnt_ref[...] = pt / jnp.maximum(pnorm, 1e-12)


def _router_block(h_ref, pn_ref, w_ref, i_ref):
    h = h_ref[...]
    hnorm = jnp.maximum(jnp.sqrt(_tree_row_sumsq(h)), 1e-12)
    hn = h / hnorm

    logits = jax.lax.dot_general(
        hn, pn_ref[...],
        (((1,), (0,)), ((), ())),
        preferred_element_type=jnp.float32,
    )
    iota_f = jax.lax.broadcasted_iota(
        jnp.int32, (_BT, _EPAD), 1).astype(jnp.float32)
    logits = jnp.where(iota_f < _NUM_EXPERTS, logits, _NEG_INF)

    m = jnp.max(logits, axis=1, keepdims=True)
    e = jnp.exp(logits - m)
    z = jnp.sum(e, axis=1, keepdims=True)
    probs = e / z

    cur = probs
    vals, ids = [], []
    for _ in range(_TOP_K):
        mx = jnp.max(cur, axis=1, keepdims=True)
        hit = cur == mx
        am = jnp.min(jnp.where(hit, iota_f, float(_EPAD)),
                     axis=1, keepdims=True)
        vals.append(mx)
        ids.append(am)
        cur = jnp.where(iota_f == am, -1.0, cur)

    w_ref[...] = jnp.concatenate(vals, axis=1)
    i_ref[...] = jnp.concatenate(ids, axis=1).astype(jnp.int32)


@jax.jit
def kernel(hidden_states, proto):
    proto_t = jnp.pad(proto, ((0, _EPAD - _NUM_EXPERTS), (0, 0))).T
    pn = pl.pallas_call(
        _proto_norm_block,
        out_shape=jax.ShapeDtypeStruct((_HIDDEN, _EPAD), jnp.float32),
    )(proto_t)

    grid = _TOKENS // _BT
    return pl.pallas_call(
        _router_block,
        grid=(grid,),
        in_specs=[
            pl.BlockSpec((_BT, _HIDDEN), lambda i: (i, 0)),
            pl.BlockSpec((_HIDDEN, _EPAD), lambda i: (0, 0)),
        ],
        out_specs=[
            pl.BlockSpec((_BT, _TOP_K), lambda i: (i, 0)),
            pl.BlockSpec((_BT, _TOP_K), lambda i: (i, 0)),
        ],
        out_shape=[
            jax.ShapeDtypeStruct((_TOKENS, _TOP_K), jnp.float32),
            jax.ShapeDtypeStruct((_TOKENS, _TOP_K), jnp.int32),
        ],
        compiler_params=pltpu.CompilerParams(
            dimension_semantics=("parallel",),
        ),
    )(hidden_states, pn)
